# baseline (speedup 1.0000x reference)
import functools

import jax
import jax.numpy as jnp
from jax import lax
from jax.experimental import pallas as pl
from jax.experimental.pallas import tpu as pltpu
from jax.experimental.pallas import tpu_sc as plsc

B = 16384
DX = 512
DY = 64
RPS = 8192
NT = 4
NC = 2
NS = 16
NW = NC * NS
BPW = B // NW
KC = 128
NCHMAX = BPW // KC + 1
LCAP = BPW + KC

_MESH = plsc.VectorSubcoreMesh(core_axis_name="c", subcore_axis_name="s")


@functools.partial(
    pl.kernel,
    mesh=_MESH,
    compiler_params=pltpu.CompilerParams(use_tc_tiling_on_sc=False, needs_layout_passes=False),
    out_type=[
        jax.ShapeDtypeStruct((B, DX), jnp.float32),
        jax.ShapeDtypeStruct((B, DY), jnp.float32),
    ],
    scratch_types=[
        pltpu.VMEM((BPW,), jnp.int32),
        pltpu.VMEM((NT, LCAP), jnp.int32),
        pltpu.VMEM((NT, LCAP), jnp.int32),
        pltpu.VMEM((NT * NCHMAX, KC), jnp.int32),
        pltpu.VMEM((KC, DX), jnp.float32),
        pltpu.VMEM((KC, DY), jnp.float32),
        pltpu.SemaphoreType.DMA,
        pltpu.SemaphoreType.DMA,
    ],
)
def _collate_kernel(x0, x1, x2, x3, y0, y1, y2, y3, idx_hbm, outx, outy,
                    idx_v, loc2, pos1, pos3, stx, sty, semg, sems):
    xt = (x0, x1, x2, x3)
    yt = (y0, y1, y2, y3)
    wid = lax.axis_index("s") * NC + lax.axis_index("c")
    base = wid * BPW
    pltpu.sync_copy(idx_hbm.at[pl.ds(base, BPW)], idx_v)

    lanes = lax.iota(jnp.int32, 16)

    def part_body(i, offs):
        iv = idx_v[pl.ds(i * 16, 16)]
        tid = lax.shift_right_logical(iv, 13)
        loc = lax.bitwise_and(iv, RPS - 1)
        pos = base + i * 16 + lanes
        new = []
        for t in range(NT):
            m = tid == t
            cum = plsc.cumsum(m.astype(jnp.int32))
            dst = offs[t] + cum - 1
            plsc.store_scatter(loc2.at[t], [dst], loc, mask=m)
            plsc.store_scatter(pos1.at[t], [dst], pos, mask=m)
            new.append(offs[t] + cum[15])
        return tuple(new)

    zero = jnp.int32(0)
    offs = lax.fori_loop(0, BPW // 16, part_body, (zero, zero, zero, zero))

    for t in range(NT):
        c = offs[t]

        @pl.when(c > 0)
        def _patch(t=t, c=c):
            fill = jnp.full((16,), c - 1, jnp.int32)
            lastl = plsc.load_gather(loc2.at[t], [fill])
            lastp = plsc.load_gather(pos1.at[t], [fill])
            for r in range(KC // 16):
                padidx = c + r * 16 + lanes
                plsc.store_scatter(loc2.at[t], [padidx], lastl)
                plsc.store_scatter(pos1.at[t], [padidx], lastp)

        nch = (c + KC - 1) // KC

        def repack(j, carry, t=t):
            for r in range(KC // 16):
                pos3[t * NCHMAX + j, pl.ds(r * 16, 16)] = (
                    pos1[t, pl.ds(j * KC + r * 16, 16)])
            return carry

        lax.fori_loop(0, nch, repack, 0)

        def gs_body(j, carry, t=t):
            ids = loc2.at[t].at[pl.ds(j * KC, KC)]
            prow = pos3.at[t * NCHMAX + j]
            pltpu.async_copy(xt[t].at[ids], stx, semg).wait()
            pltpu.async_copy(yt[t].at[ids], sty, semg).wait()
            pltpu.async_copy(stx, outx.at[prow], sems).wait()
            pltpu.async_copy(sty, outy.at[prow], sems).wait()
            return carry

        lax.fori_loop(0, nch, gs_body, 0)


def kernel(x0, x1, x2, x3, y0, y1, y2, y3, random_idx):
    bx, by = _collate_kernel(x0, x1, x2, x3, y0, y1, y2, y3,
                             random_idx.astype(jnp.int32))
    return (bx, by)


# speedup vs baseline: 1.5557x; 1.5557x over previous
import functools

import jax
import jax.numpy as jnp
from jax import lax
from jax.experimental import pallas as pl
from jax.experimental.pallas import tpu as pltpu
from jax.experimental.pallas import tpu_sc as plsc

B = 16384
DX = 512
DY = 64
RPS = 8192
NT = 4
NC = 2
NS = 16
NW = NC * NS
BPW = B // NW
KC = 128
LCAP = BPW + KC

_MESH = plsc.VectorSubcoreMesh(core_axis_name="c", subcore_axis_name="s")


@functools.partial(
    pl.kernel,
    mesh=_MESH,
    compiler_params=pltpu.CompilerParams(use_tc_tiling_on_sc=False,
                                         needs_layout_passes=False),
    out_type=[
        jax.ShapeDtypeStruct((B, DX), jnp.float32),
        jax.ShapeDtypeStruct((B, DY), jnp.float32),
    ],
    scratch_types=[
        pltpu.VMEM((BPW,), jnp.int32),
        pltpu.VMEM((NT, LCAP), jnp.int32),
        pltpu.VMEM((NT, LCAP), jnp.int32),
        pltpu.VMEM((KC, DX), jnp.float32),
        pltpu.VMEM((KC, DY), jnp.float32),
        pltpu.SemaphoreType.DMA,
        pltpu.SemaphoreType.DMA,
    ],
)
def _collate_kernel(x0, x1, x2, x3, y0, y1, y2, y3, idx_hbm, outx, outy,
                    idx_v, loc2, pos1, stx, sty, semg, sems):
    xt = (x0, x1, x2, x3)
    yt = (y0, y1, y2, y3)
    wid = lax.axis_index("s") * NC + lax.axis_index("c")
    base = wid * BPW
    pltpu.sync_copy(idx_hbm.at[pl.ds(base, BPW)], idx_v)

    lanes = lax.iota(jnp.int32, 16)
    zero = jnp.int32(0)

    def part_body(i, offs):
        iv = idx_v[pl.ds(i * 16, 16)]
        tid = lax.shift_right_logical(iv, 13)
        loc = lax.bitwise_and(iv, RPS - 1)
        pos = base + i * 16 + lanes
        new = []
        for t in range(NT):
            m = tid == t
            cum = plsc.cumsum(m.astype(jnp.int32))
            dst = offs[t] + cum - 1
            plsc.store_scatter(loc2.at[t], [dst], loc, mask=m)
            plsc.store_scatter(pos1.at[t], [dst], pos, mask=m)
            new.append(offs[t] + cum[15])
        return tuple(new)

    offs = lax.fori_loop(0, BPW // 16, part_body, (zero, zero, zero, zero))

    for t in range(NT):
        c = offs[t]

        fill = jnp.full((16,), 1, jnp.int32) * jnp.maximum(c - 1, 0)
        lastl = plsc.load_gather(loc2.at[t], [fill])
        lastp = plsc.load_gather(pos1.at[t], [fill])
        for r in range(KC // 16):
            padidx = c + r * 16 + lanes
            plsc.store_scatter(loc2.at[t], [padidx], lastl)
            plsc.store_scatter(pos1.at[t], [padidx], lastp)

        s = jnp.maximum(lax.bitwise_and(c + 7, ~jnp.int32(7)), jnp.int32(KC))
        nwin = (c + KC - 1) // KC
        last0 = s - KC

        def gs_body(w, carry, t=t, last0=last0):
            start = pl.multiple_of(jnp.minimum(w * KC, last0), 8)
            ids = loc2.at[t].at[pl.ds(start, KC)]
            prow = pos1.at[t].at[pl.ds(start, KC)]
            gx = pltpu.async_copy(xt[t].at[ids], stx, semg)
            gy = pltpu.async_copy(yt[t].at[ids], sty, semg)
            gx.wait()
            gy.wait()
            sx = pltpu.async_copy(stx, outx.at[prow], sems)
            sy = pltpu.async_copy(sty, outy.at[prow], sems)
            sx.wait()
            sy.wait()
            return carry

        lax.fori_loop(0, nwin, gs_body, 0)


def kernel(x0, x1, x2, x3, y0, y1, y2, y3, random_idx):
    bx, by = _collate_kernel(x0, x1, x2, x3, y0, y1, y2, y3,
                             random_idx.astype(jnp.int32))
    return (bx, by)
